# Initial kernel scaffold; baseline (speedup 1.0000x reference)
import jax
import jax.numpy as jnp
from jax.experimental import pallas as pl


def kernel(codes, tables, pos_emb):
    raise NotImplementedError("write your pallas kernel here")


# speedup vs baseline: 1.6408x; 1.6408x over previous
import functools

import jax
import jax.numpy as jnp
from jax import lax
from jax.experimental import pallas as pl
from jax.experimental.pallas import tpu as pltpu
from jax.experimental.pallas import tpu_sc as plsc

_PAD = 1024
_VOCAB = 1025
_LANES = 16

_NC = 2
_NS = 16
_NW = _NC * _NS

_CHUNK = 8


def _sc_body(comb_hbm, idx_hbm, out_hbm, idx_v, rows_v, out_v, sem):
    n_rows, d = out_hbm.shape
    n_src = idx_v.shape[0] // _CHUNK
    g = _CHUNK * n_src
    rows_per_w = n_rows // _NW
    steps = rows_per_w // _CHUNK
    wid = lax.axis_index("s") * _NC + lax.axis_index("c")

    def step(c, carry):
        base = wid * rows_per_w + c * _CHUNK
        ibase = pl.multiple_of(base * n_src, 8)
        pltpu.sync_copy(idx_hbm.at[pl.ds(ibase, g)], idx_v)
        pltpu.async_copy(comb_hbm.at[idx_v], rows_v, sem).wait()

        def col(j, carry2):
            off = j * _LANES
            for r in range(_CHUNK):
                acc = rows_v[r * n_src, pl.ds(off, _LANES)]
                for k in range(1, n_src):
                    acc = acc + rows_v[r * n_src + k, pl.ds(off, _LANES)]
                out_v[r, pl.ds(off, _LANES)] = acc
            return carry2

        lax.fori_loop(0, d // _LANES, col, 0)
        pltpu.sync_copy(out_v, out_hbm.at[pl.ds(base, _CHUNK)])
        return carry

    lax.fori_loop(0, steps, step, 0)


def kernel(codes, tables, pos_emb):
    B, K, T = codes.shape
    d = tables.shape[-1]
    n = B * T
    n_src = K + 1

    flat = tables.reshape(K * _VOCAB, d)
    zero_base = K * _VOCAB
    pos_base = zero_base + 8
    comb = jnp.concatenate(
        [flat, jnp.zeros((8, d), jnp.float32), pos_emb[:T]], axis=0)

    codes_t = codes.transpose(0, 2, 1).reshape(n, K)
    k_ar = jnp.arange(K, dtype=jnp.int32)[None, :]
    idx8 = jnp.where(codes_t == _PAD, zero_base + k_ar,
                     codes_t + k_ar * _VOCAB)
    tcol = (jnp.arange(n, dtype=jnp.int32) % T) + pos_base
    idx = jnp.concatenate([idx8, tcol[:, None]], axis=1)
    idx = idx.reshape(n * n_src).astype(jnp.int32)

    mesh = plsc.VectorSubcoreMesh(core_axis_name="c", subcore_axis_name="s")
    fn = functools.partial(
        pl.kernel,
        mesh=mesh,
        out_type=jax.ShapeDtypeStruct((n, d), jnp.float32),
        scratch_types=[
            pltpu.VMEM((_CHUNK * n_src,), jnp.int32),
            pltpu.VMEM((_CHUNK * n_src, d), jnp.float32),
            pltpu.VMEM((_CHUNK, d), jnp.float32),
            pltpu.SemaphoreType.DMA,
        ],
    )(_sc_body)
    out = fn(comb, idx)
    return out.reshape(B, T, d)


# speedup vs baseline: 1.7351x; 1.0574x over previous
import functools

import jax
import jax.numpy as jnp
from jax import lax
from jax.experimental import pallas as pl
from jax.experimental.pallas import tpu as pltpu
from jax.experimental.pallas import tpu_sc as plsc

_PAD = 1024
_VOCAB = 1025
_LANES = 16

_NC = 2
_NS = 16
_NW = _NC * _NS

_CHUNK = 4


def _sc_body(comb_hbm, idx_hbm, pos_hbm, out_hbm, idx_v,
             rows0, rows1, pos0, pos1, outv0, outv1,
             sr0, sr1, sp0, sp1, ss0, ss1):
    n_rows, d = out_hbm.shape
    t_len = pos_hbm.shape[0]
    k = rows0.shape[0] // _CHUNK
    g = _CHUNK * k
    rows_per_w = n_rows // _NW
    steps = rows_per_w // _CHUNK
    wid = lax.axis_index("s") * _NC + lax.axis_index("c")

    rows = (rows0, rows1)
    pos = (pos0, pos1)
    outv = (outv0, outv1)
    sem_r = (sr0, sr1)
    sem_p = (sp0, sp1)
    sem_s = (ss0, ss1)

    pltpu.sync_copy(idx_hbm.at[pl.ds(wid * rows_per_w * k, rows_per_w * k)],
                    idx_v)

    def fire(c, b):
        pltpu.async_copy(comb_hbm.at[idx_v.at[pl.ds(c * g, g)]],
                         rows[b], sem_r[b])
        base = wid * rows_per_w + c * _CHUNK
        t0 = lax.rem(base, t_len)
        pltpu.async_copy(pos_hbm.at[pl.ds(t0, _CHUNK)], pos[b], sem_p[b])

    def wait_fired(b):
        pltpu.make_async_copy(comb_hbm.at[idx_v.at[pl.ds(0, g)]],
                              rows[b], sem_r[b]).wait()
        pltpu.make_async_copy(pos_hbm.at[pl.ds(0, _CHUNK)],
                              pos[b], sem_p[b]).wait()

    def wait_store(b):
        pltpu.make_async_copy(outv[b], out_hbm.at[pl.ds(0, _CHUNK)],
                              sem_s[b]).wait()

    def accumulate(b):
        def col(j, carry):
            off = j * _LANES
            for r in range(_CHUNK):
                acc = pos[b][r, pl.ds(off, _LANES)]
                for kk in range(k):
                    acc = acc + rows[b][r * k + kk, pl.ds(off, _LANES)]
                outv[b][r, pl.ds(off, _LANES)] = acc
            return carry

        lax.fori_loop(0, d // _LANES, col, 0)

    fire(0, 0)
    fire(1, 1)

    def pair(i, carry):
        for b in range(2):
            c = 2 * i + b
            pl.when(i >= 1)(lambda b=b: wait_store(b))
            wait_fired(b)
            accumulate(b)
            base = wid * rows_per_w + c * _CHUNK
            pltpu.async_copy(outv[b], out_hbm.at[pl.ds(base, _CHUNK)],
                             sem_s[b])
            pl.when(i < steps // 2 - 1)(lambda c=c, b=b: fire(c + 2, b))
        return carry

    lax.fori_loop(0, steps // 2, pair, 0)
    wait_store(0)
    wait_store(1)


def kernel(codes, tables, pos_emb):
    B, K, T = codes.shape
    d = tables.shape[-1]
    n = B * T

    flat = tables.reshape(K * _VOCAB, d)
    zero_base = K * _VOCAB
    comb = jnp.concatenate([flat, jnp.zeros((8, d), jnp.float32)], axis=0)

    codes_t = codes.transpose(0, 2, 1).reshape(n, K)
    k_ar = jnp.arange(K, dtype=jnp.int32)[None, :]
    idx = jnp.where(codes_t == _PAD, zero_base + k_ar,
                    codes_t + k_ar * _VOCAB)
    idx = idx.reshape(n * K).astype(jnp.int32)

    mesh = plsc.VectorSubcoreMesh(core_axis_name="c", subcore_axis_name="s")
    rows_per_w = n // _NW
    fn = functools.partial(
        pl.kernel,
        mesh=mesh,
        out_type=jax.ShapeDtypeStruct((n, d), jnp.float32),
        scratch_types=[
            pltpu.VMEM((rows_per_w * K,), jnp.int32),
            pltpu.VMEM((_CHUNK * K, d), jnp.float32),
            pltpu.VMEM((_CHUNK * K, d), jnp.float32),
            pltpu.VMEM((_CHUNK, d), jnp.float32),
            pltpu.VMEM((_CHUNK, d), jnp.float32),
            pltpu.VMEM((_CHUNK, d), jnp.float32),
            pltpu.VMEM((_CHUNK, d), jnp.float32),
            pltpu.SemaphoreType.DMA,
            pltpu.SemaphoreType.DMA,
            pltpu.SemaphoreType.DMA,
            pltpu.SemaphoreType.DMA,
            pltpu.SemaphoreType.DMA,
            pltpu.SemaphoreType.DMA,
        ],
    )(_sc_body)
    out = fn(comb, idx, pos_emb[:T])
    return out.reshape(B, T, d)
